# baseline (speedup 1.0000x reference)
import functools

import jax
import jax.numpy as jnp
from jax import lax
from jax.experimental import pallas as pl
from jax.experimental.pallas import tpu as pltpu
from jax.experimental.pallas import tpu_sc as plsc

B, O, T, D, H, PO, P = 2, 10000, 160000, 128, 128, 128, 64
BT = B * T
W1B_OUT = 2 * H + PO
NC, NS = 2, 16
NW = NC * NS
EPW = BT // NW
CH = 80
NCH_G = EPW // CH
EPT = T // NS
NCH_S = EPT // CH
OPAD = 10240
ZR = OPAD // NS
CROWS = OPAD // D

_sc_mesh = plsc.VectorSubcoreMesh(core_axis_name="c", subcore_axis_name="s")


@functools.partial(
    pl.kernel,
    out_type=(jax.ShapeDtypeStruct((BT, D), jnp.float32),
              jax.ShapeDtypeStruct((BT, D), jnp.float32)),
    mesh=_sc_mesh,
    scratch_types=[
        pltpu.VMEM((CH,), jnp.int32),
        pltpu.VMEM((CH, D), jnp.float32),
        pltpu.VMEM((CH,), jnp.int32),
        pltpu.VMEM((CH, D), jnp.float32),
        pltpu.SemaphoreType.DMA,
        pltpu.SemaphoreType.DMA,
    ],
)
def _gather_k(obj_hbm, sidx_hbm, oidx_hbm, outs_hbm, outo_hbm,
              sidx_v, srows_v, oidx_v, orows_v, ssem, osem):
    wid = lax.axis_index("s") * NC + lax.axis_index("c")
    base = wid * EPW

    def chunk(i, carry):
        off = base + i * CH
        pltpu.sync_copy(sidx_hbm.at[pl.ds(off, CH)], sidx_v)
        pltpu.sync_copy(oidx_hbm.at[pl.ds(off, CH)], oidx_v)
        s_dma = pltpu.async_copy(obj_hbm.at[sidx_v], srows_v, ssem)
        o_dma = pltpu.async_copy(obj_hbm.at[oidx_v], orows_v, osem)
        s_dma.wait()
        o_dma.wait()
        pltpu.sync_copy(srows_v, outs_hbm.at[pl.ds(off, CH)])
        pltpu.sync_copy(orows_v, outo_hbm.at[pl.ds(off, CH)])
        return carry

    lax.fori_loop(0, NCH_G, chunk, 0)


@functools.partial(
    pl.kernel,
    out_type=jax.ShapeDtypeStruct((B * OPAD, D), jnp.float32),
    mesh=_sc_mesh,
    scratch_types=[
        pltpu.VMEM_SHARED((OPAD, D), jnp.float32),
        pltpu.VMEM((CH,), jnp.int32),
        pltpu.VMEM((CH, D), jnp.float32),
        pltpu.VMEM((CH,), jnp.int32),
        pltpu.VMEM((CH, D), jnp.float32),
    ],
)
def _scatter_k(sidx_hbm, oidx_hbm, cs_hbm, co_hbm, zeros_hbm, outv_hbm,
               acc, sidx_v, srows_v, oidx_v, orows_v):
    c = lax.axis_index("c")
    sid = lax.axis_index("s")
    pltpu.sync_copy(zeros_hbm.at[pl.ds(0, ZR)], acc.at[pl.ds(sid * ZR, ZR)])
    plsc.subcore_barrier()

    base = c * T + sid * EPT

    def chunk(i, carry):
        off = base + i * CH
        pltpu.sync_copy(sidx_hbm.at[pl.ds(off, CH)], sidx_v)
        pltpu.sync_copy(cs_hbm.at[pl.ds(off, CH)], srows_v)
        pltpu.sync_copy(srows_v, acc.at[sidx_v], add=True)
        pltpu.sync_copy(oidx_hbm.at[pl.ds(off, CH)], oidx_v)
        pltpu.sync_copy(co_hbm.at[pl.ds(off, CH)], orows_v)
        pltpu.sync_copy(orows_v, acc.at[oidx_v], add=True)
        return carry

    lax.fori_loop(0, NCH_S, chunk, 0)
    plsc.subcore_barrier()
    pltpu.sync_copy(acc.at[pl.ds(sid * ZR, ZR)],
                    outv_hbm.at[pl.ds(c * OPAD + sid * ZR, ZR)])


MLP_TILE = 512


def _mlp_body(s_ref, p_ref, o_ref, scal_ref,
              w1s_ref, w1p_ref, w1o_ref, b1a_ref, w1b_ref, b1b_ref, ptw_ref,
              newp_ref, cs_ref, co_ref, cnt_ref):
    s = s_ref[...]
    pv = p_ref[...]
    o = o_ref[...]
    eye5 = (lax.broadcasted_iota(jnp.int32, (5, 5), 0)
            == lax.broadcasted_iota(jnp.int32, (5, 5), 1)).astype(jnp.float32)
    scal_t = lax.dot_general(scal_ref[...], eye5, (((0,), (0,)), ((), ())),
                             precision=lax.Precision.HIGHEST,
                             preferred_element_type=jnp.float32)
    tt = scal_t[:, 0:1]
    pid = scal_t[:, 1:2]
    w = scal_t[:, 2:3]
    sif = scal_t[:, 3:4]
    oif = scal_t[:, 4:5]
    h = (jnp.dot(s, w1s_ref[...], preferred_element_type=jnp.float32)
         + jnp.dot(pv, w1p_ref[...], preferred_element_type=jnp.float32)
         + jnp.dot(o, w1o_ref[...], preferred_element_type=jnp.float32)
         + b1a_ref[...])
    h = jnp.maximum(h, 0.0)
    new_t = jnp.dot(h, w1b_ref[...], preferred_element_type=jnp.float32) + b1b_ref[...]
    new_t = jnp.maximum(new_t, 0.0)

    ptp = jax.nn.sigmoid(ptw_ref[...])
    lanes = lax.broadcasted_iota(jnp.int32, (MLP_TILE, P), 1).astype(jnp.float32)
    onehot = (lanes == pid).astype(jnp.float32)
    conf_t = jnp.sum(onehot * ptp, axis=1, keepdims=True)
    conf = jnp.where(tt == 0.0, 1.0, conf_t)
    cfw = conf * w

    newp_ref[...] = new_t[:, D:2 * D] * conf
    cs_ref[...] = new_t[:, :D] * cfw
    co_ref[...] = new_t[:, 2 * D:] * cfw

    i = pl.program_id(0)
    rows = lax.broadcasted_iota(jnp.int32, (MLP_TILE, 1), 0).astype(jnp.float32)
    batch = jnp.where(rows + i * MLP_TILE >= T, 1.0, 0.0)
    lanes160 = lax.broadcasted_iota(jnp.int32, (MLP_TILE, 2 * CROWS), 1).astype(jnp.float32)
    lanes128 = lax.broadcasted_iota(jnp.int32, (MLP_TILE, D), 1).astype(jnp.float32)

    def hist(node_f):
        hi = jnp.floor(node_f * (1.0 / 128.0))
        lo = node_f - hi * 128.0
        hirow = hi + batch * CROWS
        a = jnp.where(lanes160 == hirow, cfw, 0.0)
        bm = (lanes128 == lo).astype(jnp.float32)
        return lax.dot_general(a, bm, (((0,), (0,)), ((), ())),
                               precision=lax.Precision.HIGHEST,
                               preferred_element_type=jnp.float32)

    contrib = hist(sif) + hist(oif)

    @pl.when(i == 0)
    def _():
        cnt_ref[...] = jnp.zeros_like(cnt_ref)

    cnt_ref[...] += contrib


_mlp_call = pl.pallas_call(
    _mlp_body,
    grid=(BT // MLP_TILE,),
    in_specs=[
        pl.BlockSpec((MLP_TILE, D), lambda i: (i, 0)),
        pl.BlockSpec((MLP_TILE, D), lambda i: (i, 0)),
        pl.BlockSpec((MLP_TILE, D), lambda i: (i, 0)),
        pl.BlockSpec((5, MLP_TILE), lambda i: (0, i)),
        pl.BlockSpec((D, H), lambda i: (0, 0)),
        pl.BlockSpec((D, H), lambda i: (0, 0)),
        pl.BlockSpec((D, H), lambda i: (0, 0)),
        pl.BlockSpec((1, H), lambda i: (0, 0)),
        pl.BlockSpec((H, W1B_OUT), lambda i: (0, 0)),
        pl.BlockSpec((1, W1B_OUT), lambda i: (0, 0)),
        pl.BlockSpec((1, P), lambda i: (0, 0)),
    ],
    out_specs=[
        pl.BlockSpec((MLP_TILE, D), lambda i: (i, 0)),
        pl.BlockSpec((MLP_TILE, D), lambda i: (i, 0)),
        pl.BlockSpec((MLP_TILE, D), lambda i: (i, 0)),
        pl.BlockSpec((2 * CROWS, D), lambda i: (0, 0)),
    ],
    out_shape=[
        jax.ShapeDtypeStruct((BT, D), jnp.float32),
        jax.ShapeDtypeStruct((BT, D), jnp.float32),
        jax.ShapeDtypeStruct((BT, D), jnp.float32),
        jax.ShapeDtypeStruct((2 * CROWS, D), jnp.float32),
    ],
    compiler_params=pltpu.CompilerParams(dimension_semantics=("arbitrary",)),
)


OUT_TILE = 2048


def _out_body(pp_ref, cnt_ref, w2a_ref, b2a_ref, w2b_ref, b2b_ref, out_ref):
    pooled = pp_ref[...]
    cnt = cnt_ref[...]
    denom = jnp.where(cnt > 0.0, cnt, 1.0)
    pn = pooled / denom
    h2 = jnp.maximum(
        jnp.dot(pn, w2a_ref[...], preferred_element_type=jnp.float32)
        + b2a_ref[...], 0.0)
    out_ref[...] = jnp.maximum(
        jnp.dot(h2, w2b_ref[...], preferred_element_type=jnp.float32)
        + b2b_ref[...], 0.0)


_out_call = pl.pallas_call(
    _out_body,
    grid=(B * OPAD // OUT_TILE,),
    in_specs=[
        pl.BlockSpec((OUT_TILE, D), lambda i: (i, 0)),
        pl.BlockSpec((OUT_TILE, 1), lambda i: (i, 0)),
        pl.BlockSpec((H, H), lambda i: (0, 0)),
        pl.BlockSpec((1, H), lambda i: (0, 0)),
        pl.BlockSpec((H, D), lambda i: (0, 0)),
        pl.BlockSpec((1, D), lambda i: (0, 0)),
    ],
    out_specs=pl.BlockSpec((OUT_TILE, D), lambda i: (i, 0)),
    out_shape=jax.ShapeDtypeStruct((B * OPAD, D), jnp.float32),
    compiler_params=pltpu.CompilerParams(dimension_semantics=("arbitrary",)),
)


def kernel(obj_vecs, pred_vecs, edges, pred_indicators, triplet_type,
           predicate_ids, W1a, b1a, W1b, b1b, W2a, b2a, W2b, b2b, ptw):
    s_idx = edges[:, :, 0]
    o_idx = edges[:, :, 1]
    boff = (jnp.arange(B, dtype=jnp.int32) * O)[:, None]
    sflat_g = (s_idx + boff).reshape(BT)
    oflat_g = (o_idx + boff).reshape(BT)
    obj_flat = obj_vecs.reshape(B * O, D)

    cur_s, cur_o = _gather_k(obj_flat, sflat_g, oflat_g)

    pred_flat = pred_vecs.reshape(BT, D)
    scal = jnp.stack([
        triplet_type.astype(jnp.float32).reshape(BT),
        predicate_ids.astype(jnp.float32).reshape(BT),
        pred_indicators.astype(jnp.float32).reshape(BT),
        s_idx.astype(jnp.float32).reshape(BT),
        o_idx.astype(jnp.float32).reshape(BT),
    ])

    new_p, cs, co, cnt = _mlp_call(
        cur_s, pred_flat, cur_o, scal,
        W1a[:D], W1a[D:2 * D], W1a[2 * D:], b1a.reshape(1, H),
        W1b, b1b.reshape(1, W1B_OUT), ptw.reshape(1, P))

    zeros = jnp.zeros((ZR, D), jnp.float32)
    pp = _scatter_k(s_idx.reshape(BT), o_idx.reshape(BT), cs, co, zeros)

    cnt_col = cnt.reshape(B * OPAD, 1)
    new_obj = _out_call(pp, cnt_col, W2a, b2a.reshape(1, H),
                        W2b, b2b.reshape(1, D))
    return new_obj.reshape(B, OPAD, D)[:, :O], new_p.reshape(B, T, D)


# speedup vs baseline: 1.1243x; 1.1243x over previous
import functools

import jax
import jax.numpy as jnp
from jax import lax
from jax.experimental import pallas as pl
from jax.experimental.pallas import tpu as pltpu
from jax.experimental.pallas import tpu_sc as plsc

B, O, T, D, H, PO, P = 2, 10000, 160000, 128, 128, 128, 64
BT = B * T
W1B_OUT = 2 * H + PO
NC, NS = 2, 16
NW = NC * NS
EPW = BT // NW
CH = 80
NCH_G = EPW // CH
EPT = T // NS
NCH_S = EPT // CH
OPAD = 10240
ZR = OPAD // NS
CROWS = OPAD // D

_sc_mesh = plsc.VectorSubcoreMesh(core_axis_name="c", subcore_axis_name="s")


@functools.partial(
    pl.kernel,
    out_type=(jax.ShapeDtypeStruct((BT, D), jnp.float32),
              jax.ShapeDtypeStruct((BT, D), jnp.float32)),
    mesh=_sc_mesh,
    scratch_types=[
        pltpu.VMEM((CH,), jnp.int32),
        pltpu.VMEM((CH, D), jnp.float32),
        pltpu.VMEM((CH,), jnp.int32),
        pltpu.VMEM((CH, D), jnp.float32),
        pltpu.SemaphoreType.DMA,
        pltpu.SemaphoreType.DMA,
    ],
)
def _gather_k(obj_hbm, sidx_hbm, oidx_hbm, outs_hbm, outo_hbm,
              sidx_v, srows_v, oidx_v, orows_v, ssem, osem):
    wid = lax.axis_index("s") * NC + lax.axis_index("c")
    base = wid * EPW

    def chunk(i, carry):
        off = base + i * CH
        pltpu.sync_copy(sidx_hbm.at[pl.ds(off, CH)], sidx_v)
        pltpu.sync_copy(oidx_hbm.at[pl.ds(off, CH)], oidx_v)
        s_dma = pltpu.async_copy(obj_hbm.at[sidx_v], srows_v, ssem)
        o_dma = pltpu.async_copy(obj_hbm.at[oidx_v], orows_v, osem)
        s_dma.wait()
        o_dma.wait()
        pltpu.sync_copy(srows_v, outs_hbm.at[pl.ds(off, CH)])
        pltpu.sync_copy(orows_v, outo_hbm.at[pl.ds(off, CH)])
        return carry

    lax.fori_loop(0, NCH_G, chunk, 0)


@functools.partial(
    pl.kernel,
    out_type=jax.ShapeDtypeStruct((B * OPAD, D), jnp.float32),
    mesh=_sc_mesh,
    scratch_types=[
        pltpu.VMEM_SHARED((OPAD, D), jnp.float32),
        pltpu.VMEM((CH,), jnp.int32),
        pltpu.VMEM((CH, D), jnp.float32),
        pltpu.VMEM((CH,), jnp.int32),
        pltpu.VMEM((CH, D), jnp.float32),
    ],
)
def _scatter_k(sidx_hbm, oidx_hbm, cs_hbm, co_hbm, zeros_hbm, outv_hbm,
               acc, sidx_v, srows_v, oidx_v, orows_v):
    c = lax.axis_index("c")
    sid = lax.axis_index("s")
    pltpu.sync_copy(zeros_hbm.at[pl.ds(0, ZR)], acc.at[pl.ds(sid * ZR, ZR)])
    plsc.subcore_barrier()

    base = c * T + sid * EPT

    def chunk(i, carry):
        off = base + i * CH
        pltpu.sync_copy(sidx_hbm.at[pl.ds(off, CH)], sidx_v)
        pltpu.sync_copy(cs_hbm.at[pl.ds(off, CH)], srows_v)
        pltpu.sync_copy(srows_v, acc.at[sidx_v], add=True)
        pltpu.sync_copy(oidx_hbm.at[pl.ds(off, CH)], oidx_v)
        pltpu.sync_copy(co_hbm.at[pl.ds(off, CH)], orows_v)
        pltpu.sync_copy(orows_v, acc.at[oidx_v], add=True)
        return carry

    lax.fori_loop(0, NCH_S, chunk, 0)
    plsc.subcore_barrier()
    pltpu.sync_copy(acc.at[pl.ds(sid * ZR, ZR)],
                    outv_hbm.at[pl.ds(c * OPAD + sid * ZR, ZR)])


MLP_TILE = 512


def _mlp_body(s_ref, p_ref, o_ref, scal_ref,
              w1s_ref, w1p_ref, w1o_ref, b1a_ref, w1b_ref, b1b_ref, ptw_ref,
              newp_ref, cs_ref, co_ref, cnt_ref):
    s = s_ref[...]
    pv = p_ref[...]
    o = o_ref[...]
    eye5 = (lax.broadcasted_iota(jnp.int32, (5, 5), 0)
            == lax.broadcasted_iota(jnp.int32, (5, 5), 1)).astype(jnp.float32)
    scal_t = lax.dot_general(scal_ref[...], eye5, (((0,), (0,)), ((), ())),
                             precision=lax.Precision.HIGHEST,
                             preferred_element_type=jnp.float32)
    tt = scal_t[:, 0:1]
    pid = scal_t[:, 1:2]
    w = scal_t[:, 2:3]
    sif = scal_t[:, 3:4]
    oif = scal_t[:, 4:5]
    h = (jnp.dot(s, w1s_ref[...], preferred_element_type=jnp.float32)
         + jnp.dot(pv, w1p_ref[...], preferred_element_type=jnp.float32)
         + jnp.dot(o, w1o_ref[...], preferred_element_type=jnp.float32)
         + b1a_ref[...])
    h = jnp.maximum(h, 0.0)
    new_t = jnp.dot(h, w1b_ref[...], preferred_element_type=jnp.float32) + b1b_ref[...]
    new_t = jnp.maximum(new_t, 0.0)

    ptp = jax.nn.sigmoid(ptw_ref[...])
    lanes = lax.broadcasted_iota(jnp.int32, (MLP_TILE, P), 1).astype(jnp.float32)
    onehot = (lanes == pid).astype(jnp.float32)
    conf_t = jnp.sum(onehot * ptp, axis=1, keepdims=True)
    conf = jnp.where(tt == 0.0, 1.0, conf_t)
    cfw = conf * w

    newp_ref[...] = new_t[:, D:2 * D] * conf
    cs_ref[...] = new_t[:, :D] * cfw
    co_ref[...] = new_t[:, 2 * D:] * cfw

    i = pl.program_id(0)
    rows = lax.broadcasted_iota(jnp.int32, (MLP_TILE, 1), 0).astype(jnp.float32)
    batch = jnp.where(rows + i * MLP_TILE >= T, 1.0, 0.0)
    lanes160 = lax.broadcasted_iota(jnp.int32, (MLP_TILE, 2 * CROWS), 1).astype(jnp.float32)
    lanes128 = lax.broadcasted_iota(jnp.int32, (MLP_TILE, D), 1).astype(jnp.float32)

    def hist(node_f):
        hi = jnp.floor(node_f * (1.0 / 128.0))
        lo = node_f - hi * 128.0
        hirow = hi + batch * CROWS
        a = jnp.where(lanes160 == hirow, cfw, 0.0)
        bm = (lanes128 == lo).astype(jnp.float32)
        return lax.dot_general(a, bm, (((0,), (0,)), ((), ())),
                               preferred_element_type=jnp.float32)

    contrib = hist(sif) + hist(oif)

    @pl.when(i == 0)
    def _():
        cnt_ref[...] = jnp.zeros_like(cnt_ref)

    cnt_ref[...] += contrib


_mlp_call = pl.pallas_call(
    _mlp_body,
    grid=(BT // MLP_TILE,),
    in_specs=[
        pl.BlockSpec((MLP_TILE, D), lambda i: (i, 0)),
        pl.BlockSpec((MLP_TILE, D), lambda i: (i, 0)),
        pl.BlockSpec((MLP_TILE, D), lambda i: (i, 0)),
        pl.BlockSpec((5, MLP_TILE), lambda i: (0, i)),
        pl.BlockSpec((D, H), lambda i: (0, 0)),
        pl.BlockSpec((D, H), lambda i: (0, 0)),
        pl.BlockSpec((D, H), lambda i: (0, 0)),
        pl.BlockSpec((1, H), lambda i: (0, 0)),
        pl.BlockSpec((H, W1B_OUT), lambda i: (0, 0)),
        pl.BlockSpec((1, W1B_OUT), lambda i: (0, 0)),
        pl.BlockSpec((1, P), lambda i: (0, 0)),
    ],
    out_specs=[
        pl.BlockSpec((MLP_TILE, D), lambda i: (i, 0)),
        pl.BlockSpec((MLP_TILE, D), lambda i: (i, 0)),
        pl.BlockSpec((MLP_TILE, D), lambda i: (i, 0)),
        pl.BlockSpec((2 * CROWS, D), lambda i: (0, 0)),
    ],
    out_shape=[
        jax.ShapeDtypeStruct((BT, D), jnp.float32),
        jax.ShapeDtypeStruct((BT, D), jnp.float32),
        jax.ShapeDtypeStruct((BT, D), jnp.float32),
        jax.ShapeDtypeStruct((2 * CROWS, D), jnp.float32),
    ],
    compiler_params=pltpu.CompilerParams(dimension_semantics=("arbitrary",)),
)


OUT_TILE = 2048


def _out_body(pp_ref, cnt_ref, w2a_ref, b2a_ref, w2b_ref, b2b_ref, out_ref):
    pooled = pp_ref[...]
    cnt = cnt_ref[...]
    denom = jnp.where(cnt > 0.0, cnt, 1.0)
    pn = pooled / denom
    h2 = jnp.maximum(
        jnp.dot(pn, w2a_ref[...], preferred_element_type=jnp.float32)
        + b2a_ref[...], 0.0)
    out_ref[...] = jnp.maximum(
        jnp.dot(h2, w2b_ref[...], preferred_element_type=jnp.float32)
        + b2b_ref[...], 0.0)


_out_call = pl.pallas_call(
    _out_body,
    grid=(B * OPAD // OUT_TILE,),
    in_specs=[
        pl.BlockSpec((OUT_TILE, D), lambda i: (i, 0)),
        pl.BlockSpec((OUT_TILE, 1), lambda i: (i, 0)),
        pl.BlockSpec((H, H), lambda i: (0, 0)),
        pl.BlockSpec((1, H), lambda i: (0, 0)),
        pl.BlockSpec((H, D), lambda i: (0, 0)),
        pl.BlockSpec((1, D), lambda i: (0, 0)),
    ],
    out_specs=pl.BlockSpec((OUT_TILE, D), lambda i: (i, 0)),
    out_shape=jax.ShapeDtypeStruct((B * OPAD, D), jnp.float32),
    compiler_params=pltpu.CompilerParams(dimension_semantics=("arbitrary",)),
)


def kernel(obj_vecs, pred_vecs, edges, pred_indicators, triplet_type,
           predicate_ids, W1a, b1a, W1b, b1b, W2a, b2a, W2b, b2b, ptw):
    s_idx = edges[:, :, 0]
    o_idx = edges[:, :, 1]
    boff = (jnp.arange(B, dtype=jnp.int32) * O)[:, None]
    sflat_g = (s_idx + boff).reshape(BT)
    oflat_g = (o_idx + boff).reshape(BT)
    obj_flat = obj_vecs.reshape(B * O, D)

    cur_s, cur_o = _gather_k(obj_flat, sflat_g, oflat_g)

    pred_flat = pred_vecs.reshape(BT, D)
    scal = jnp.stack([
        triplet_type.astype(jnp.float32).reshape(BT),
        predicate_ids.astype(jnp.float32).reshape(BT),
        pred_indicators.astype(jnp.float32).reshape(BT),
        s_idx.astype(jnp.float32).reshape(BT),
        o_idx.astype(jnp.float32).reshape(BT),
    ])

    new_p, cs, co, cnt = _mlp_call(
        cur_s, pred_flat, cur_o, scal,
        W1a[:D], W1a[D:2 * D], W1a[2 * D:], b1a.reshape(1, H),
        W1b, b1b.reshape(1, W1B_OUT), ptw.reshape(1, P))

    zeros = jnp.zeros((ZR, D), jnp.float32)
    pp = _scatter_k(s_idx.reshape(BT), o_idx.reshape(BT), cs, co, zeros)

    cnt_col = cnt.reshape(B * OPAD, 1)
    new_obj = _out_call(pp, cnt_col, W2a, b2a.reshape(1, H),
                        W2b, b2b.reshape(1, D))
    return new_obj.reshape(B, OPAD, D)[:, :O], new_p.reshape(B, T, D)


# speedup vs baseline: 1.2947x; 1.1516x over previous
import functools

import jax
import jax.numpy as jnp
from jax import lax
from jax.experimental import pallas as pl
from jax.experimental.pallas import tpu as pltpu
from jax.experimental.pallas import tpu_sc as plsc

B, O, T, D, H, PO, P = 2, 10000, 160000, 128, 128, 128, 64
BT = B * T
W1B_OUT = 2 * H + PO
NC, NS = 2, 16
NW = NC * NS
EPW = BT // NW
CH = 80
NCH_G = EPW // CH
EPT = T // NS
NCH_S = EPT // CH
OPAD = 10240
ZR = OPAD // NS
CROWS = OPAD // D

_sc_mesh = plsc.VectorSubcoreMesh(core_axis_name="c", subcore_axis_name="s")


@functools.partial(
    pl.kernel,
    out_type=(jax.ShapeDtypeStruct((BT, D), jnp.float32),
              jax.ShapeDtypeStruct((BT, D), jnp.float32)),
    mesh=_sc_mesh,
    scratch_types=[
        pltpu.VMEM((CH,), jnp.int32),
        pltpu.VMEM((CH, D), jnp.float32),
        pltpu.VMEM((CH,), jnp.int32),
        pltpu.VMEM((CH, D), jnp.float32),
        pltpu.SemaphoreType.DMA,
        pltpu.SemaphoreType.DMA,
    ],
)
def _gather_k(obj_hbm, sidx_hbm, oidx_hbm, outs_hbm, outo_hbm,
              sidx_v, srows_v, oidx_v, orows_v, ssem, osem):
    wid = lax.axis_index("s") * NC + lax.axis_index("c")
    base = wid * EPW

    def chunk(i, carry):
        off = base + i * CH
        pltpu.sync_copy(sidx_hbm.at[pl.ds(off, CH)], sidx_v)
        pltpu.sync_copy(oidx_hbm.at[pl.ds(off, CH)], oidx_v)
        s_dma = pltpu.async_copy(obj_hbm.at[sidx_v], srows_v, ssem)
        o_dma = pltpu.async_copy(obj_hbm.at[oidx_v], orows_v, osem)
        s_dma.wait()
        o_dma.wait()
        pltpu.sync_copy(srows_v, outs_hbm.at[pl.ds(off, CH)])
        pltpu.sync_copy(orows_v, outo_hbm.at[pl.ds(off, CH)])
        return carry

    lax.fori_loop(0, NCH_G, chunk, 0)


@functools.partial(
    pl.kernel,
    out_type=jax.ShapeDtypeStruct((B * OPAD, D), jnp.float32),
    mesh=_sc_mesh,
    scratch_types=[
        pltpu.VMEM_SHARED((OPAD, D), jnp.float32),
        pltpu.VMEM((CH,), jnp.int32),
        pltpu.VMEM((CH, D), jnp.float32),
        pltpu.VMEM((CH,), jnp.int32),
        pltpu.VMEM((CH, D), jnp.float32),
    ],
)
def _scatter_k(sidx_hbm, oidx_hbm, cs_hbm, co_hbm, zeros_hbm, outv_hbm,
               acc, sidx_v, srows_v, oidx_v, orows_v):
    c = lax.axis_index("c")
    sid = lax.axis_index("s")
    pltpu.sync_copy(zeros_hbm.at[pl.ds(0, ZR)], acc.at[pl.ds(sid * ZR, ZR)])
    plsc.subcore_barrier()

    base = c * T + sid * EPT

    def chunk(i, carry):
        off = base + i * CH
        pltpu.sync_copy(sidx_hbm.at[pl.ds(off, CH)], sidx_v)
        pltpu.sync_copy(cs_hbm.at[pl.ds(off, CH)], srows_v)
        pltpu.sync_copy(srows_v, acc.at[sidx_v], add=True)
        pltpu.sync_copy(oidx_hbm.at[pl.ds(off, CH)], oidx_v)
        pltpu.sync_copy(co_hbm.at[pl.ds(off, CH)], orows_v)
        pltpu.sync_copy(orows_v, acc.at[oidx_v], add=True)
        return carry

    lax.fori_loop(0, NCH_S, chunk, 0)
    plsc.subcore_barrier()
    pltpu.sync_copy(acc.at[pl.ds(sid * ZR, ZR)],
                    outv_hbm.at[pl.ds(c * OPAD + sid * ZR, ZR)])


MLP_TILE = 512


def _prep_body(scal_ref, ptw_ref, ccf_ref, cnt_ref):
    i = pl.program_id(0)
    tt = scal_ref[0:1, :]
    pid = scal_ref[1:2, :]
    w = scal_ref[2:3, :]
    sif = scal_ref[3:4, :]
    oif = scal_ref[4:5, :]

    ptp = jax.nn.sigmoid(ptw_ref[...])
    sub64 = lax.broadcasted_iota(jnp.int32, (P, MLP_TILE), 0).astype(jnp.float32)
    onehot = (sub64 == pid).astype(jnp.float32)
    conf_t = jnp.dot(ptp, onehot, preferred_element_type=jnp.float32)
    conf = jnp.where(tt == 0.0, 1.0, conf_t)
    cfw = conf * w
    ccf_ref[0:1, :] = conf
    ccf_ref[1:2, :] = cfw

    lane = lax.broadcasted_iota(jnp.int32, (1, MLP_TILE), 1).astype(jnp.float32)
    batch = jnp.where(lane + i * MLP_TILE >= T, 1.0, 0.0)
    sub128 = lax.broadcasted_iota(jnp.int32, (D, MLP_TILE), 0).astype(jnp.float32)
    sub160 = lax.broadcasted_iota(jnp.int32, (2 * CROWS, MLP_TILE), 0).astype(jnp.float32)

    def hist(node_f):
        hi = jnp.floor(node_f * (1.0 / 128.0))
        lo = node_f - hi * 128.0
        hirow = hi + batch * CROWS
        a = jnp.where(sub128 == lo, cfw, 0.0)
        bm = (sub160 == hirow).astype(jnp.float32)
        return lax.dot_general(bm, a, (((1,), (1,)), ((), ())),
                               preferred_element_type=jnp.float32)

    contrib = hist(sif) + hist(oif)

    @pl.when(i == 0)
    def _():
        cnt_ref[...] = jnp.zeros_like(cnt_ref)

    cnt_ref[...] += contrib


_prep_call = pl.pallas_call(
    _prep_body,
    grid=(BT // MLP_TILE,),
    in_specs=[
        pl.BlockSpec((5, MLP_TILE), lambda i: (0, i)),
        pl.BlockSpec((1, P), lambda i: (0, 0)),
    ],
    out_specs=[
        pl.BlockSpec((2, MLP_TILE), lambda i: (0, i)),
        pl.BlockSpec((2 * CROWS, D), lambda i: (0, 0)),
    ],
    out_shape=[
        jax.ShapeDtypeStruct((2, BT), jnp.float32),
        jax.ShapeDtypeStruct((2 * CROWS, D), jnp.float32),
    ],
    compiler_params=pltpu.CompilerParams(dimension_semantics=("arbitrary",)),
)


def _mlp_body(s_ref, p_ref, o_ref, ccf_ref,
              w1s_ref, w1p_ref, w1o_ref, b1a_ref, w1b_ref, b1b_ref,
              newp_ref, cs_ref, co_ref):
    s = s_ref[...]
    pv = p_ref[...]
    o = o_ref[...]
    eye2 = (lax.broadcasted_iota(jnp.int32, (2, 2), 0)
            == lax.broadcasted_iota(jnp.int32, (2, 2), 1)).astype(jnp.float32)
    ccf_t = lax.dot_general(ccf_ref[...], eye2, (((0,), (0,)), ((), ())),
                            precision=lax.Precision.HIGHEST,
                            preferred_element_type=jnp.float32)
    conf = ccf_t[:, 0:1]
    cfw = ccf_t[:, 1:2]
    h = (jnp.dot(s, w1s_ref[...], preferred_element_type=jnp.float32)
         + jnp.dot(pv, w1p_ref[...], preferred_element_type=jnp.float32)
         + jnp.dot(o, w1o_ref[...], preferred_element_type=jnp.float32)
         + b1a_ref[...])
    h = jnp.maximum(h, 0.0)
    new_t = jnp.dot(h, w1b_ref[...], preferred_element_type=jnp.float32) + b1b_ref[...]
    new_t = jnp.maximum(new_t, 0.0)

    newp_ref[...] = new_t[:, D:2 * D] * conf
    cs_ref[...] = new_t[:, :D] * cfw
    co_ref[...] = new_t[:, 2 * D:] * cfw


_mlp_call = pl.pallas_call(
    _mlp_body,
    grid=(BT // MLP_TILE,),
    in_specs=[
        pl.BlockSpec((MLP_TILE, D), lambda i: (i, 0)),
        pl.BlockSpec((MLP_TILE, D), lambda i: (i, 0)),
        pl.BlockSpec((MLP_TILE, D), lambda i: (i, 0)),
        pl.BlockSpec((2, MLP_TILE), lambda i: (0, i)),
        pl.BlockSpec((D, H), lambda i: (0, 0)),
        pl.BlockSpec((D, H), lambda i: (0, 0)),
        pl.BlockSpec((D, H), lambda i: (0, 0)),
        pl.BlockSpec((1, H), lambda i: (0, 0)),
        pl.BlockSpec((H, W1B_OUT), lambda i: (0, 0)),
        pl.BlockSpec((1, W1B_OUT), lambda i: (0, 0)),
    ],
    out_specs=[
        pl.BlockSpec((MLP_TILE, D), lambda i: (i, 0)),
        pl.BlockSpec((MLP_TILE, D), lambda i: (i, 0)),
        pl.BlockSpec((MLP_TILE, D), lambda i: (i, 0)),
    ],
    out_shape=[
        jax.ShapeDtypeStruct((BT, D), jnp.float32),
        jax.ShapeDtypeStruct((BT, D), jnp.float32),
        jax.ShapeDtypeStruct((BT, D), jnp.float32),
    ],
    compiler_params=pltpu.CompilerParams(dimension_semantics=("arbitrary",)),
)


OUT_TILE = 2048


def _out_body(pp_ref, cnt_ref, w2a_ref, b2a_ref, w2b_ref, b2b_ref, out_ref):
    pooled = pp_ref[...]
    cnt = cnt_ref[...]
    denom = jnp.where(cnt > 0.0, cnt, 1.0)
    pn = pooled / denom
    h2 = jnp.maximum(
        jnp.dot(pn, w2a_ref[...], preferred_element_type=jnp.float32)
        + b2a_ref[...], 0.0)
    out_ref[...] = jnp.maximum(
        jnp.dot(h2, w2b_ref[...], preferred_element_type=jnp.float32)
        + b2b_ref[...], 0.0)


_out_call = pl.pallas_call(
    _out_body,
    grid=(B * OPAD // OUT_TILE,),
    in_specs=[
        pl.BlockSpec((OUT_TILE, D), lambda i: (i, 0)),
        pl.BlockSpec((OUT_TILE, 1), lambda i: (i, 0)),
        pl.BlockSpec((H, H), lambda i: (0, 0)),
        pl.BlockSpec((1, H), lambda i: (0, 0)),
        pl.BlockSpec((H, D), lambda i: (0, 0)),
        pl.BlockSpec((1, D), lambda i: (0, 0)),
    ],
    out_specs=pl.BlockSpec((OUT_TILE, D), lambda i: (i, 0)),
    out_shape=jax.ShapeDtypeStruct((B * OPAD, D), jnp.float32),
    compiler_params=pltpu.CompilerParams(dimension_semantics=("arbitrary",)),
)


def kernel(obj_vecs, pred_vecs, edges, pred_indicators, triplet_type,
           predicate_ids, W1a, b1a, W1b, b1b, W2a, b2a, W2b, b2b, ptw):
    s_idx = edges[:, :, 0]
    o_idx = edges[:, :, 1]
    boff = (jnp.arange(B, dtype=jnp.int32) * O)[:, None]
    sflat_g = (s_idx + boff).reshape(BT)
    oflat_g = (o_idx + boff).reshape(BT)
    obj_flat = obj_vecs.reshape(B * O, D)

    cur_s, cur_o = _gather_k(obj_flat, sflat_g, oflat_g)

    pred_flat = pred_vecs.reshape(BT, D)
    scal = jnp.stack([
        triplet_type.astype(jnp.float32).reshape(BT),
        predicate_ids.astype(jnp.float32).reshape(BT),
        pred_indicators.astype(jnp.float32).reshape(BT),
        s_idx.astype(jnp.float32).reshape(BT),
        o_idx.astype(jnp.float32).reshape(BT),
    ])

    ccf, cnt = _prep_call(scal, ptw.reshape(1, P))

    new_p, cs, co = _mlp_call(
        cur_s, pred_flat, cur_o, ccf,
        W1a[:D], W1a[D:2 * D], W1a[2 * D:], b1a.reshape(1, H),
        W1b, b1b.reshape(1, W1B_OUT))

    zeros = jnp.zeros((ZR, D), jnp.float32)
    pp = _scatter_k(s_idx.reshape(BT), o_idx.reshape(BT), cs, co, zeros)

    cnt_col = cnt.reshape(B * OPAD, 1)
    new_obj = _out_call(pp, cnt_col, W2a, b2a.reshape(1, H),
                        W2b, b2b.reshape(1, D))
    return new_obj.reshape(B, OPAD, D)[:, :O], new_p.reshape(B, T, D)


# speedup vs baseline: 1.3842x; 1.0691x over previous
import functools

import jax
import jax.numpy as jnp
from jax import lax
from jax.experimental import pallas as pl
from jax.experimental.pallas import tpu as pltpu
from jax.experimental.pallas import tpu_sc as plsc

B, O, T, D, H, PO, P = 2, 10000, 160000, 128, 128, 128, 64
BT = B * T
W1B_OUT = 2 * H + PO
NC, NS = 2, 16
NW = NC * NS
EPW = T // NW
CH = 80
NCH = EPW // CH
TAIL = EPW - NCH * CH
OPAD = 10240
ZR = OPAD // NS
CROWS = OPAD // D

_sc_mesh = plsc.VectorSubcoreMesh(core_axis_name="c", subcore_axis_name="s")


@functools.partial(
    pl.kernel,
    out_type=(jax.ShapeDtypeStruct((T, D), jnp.float32),
              jax.ShapeDtypeStruct((T, D), jnp.float32)),
    mesh=_sc_mesh,
    scratch_types=[
        pltpu.VMEM((CH,), jnp.int32),
        pltpu.VMEM((CH, D), jnp.float32),
        pltpu.VMEM((CH,), jnp.int32),
        pltpu.VMEM((CH, D), jnp.float32),
        pltpu.VMEM((TAIL,), jnp.int32),
        pltpu.VMEM((TAIL, D), jnp.float32),
        pltpu.VMEM((TAIL,), jnp.int32),
        pltpu.VMEM((TAIL, D), jnp.float32),
        pltpu.SemaphoreType.DMA,
        pltpu.SemaphoreType.DMA,
    ],
)
def _gather_k(obj_hbm, sidx_hbm, oidx_hbm, outs_hbm, outo_hbm,
              sidx_v, srows_v, oidx_v, orows_v,
              sidx_t, srows_t, oidx_t, orows_t, ssem, osem):
    wid = lax.axis_index("s") * NC + lax.axis_index("c")
    base = wid * EPW

    def chunk(i, carry):
        off = base + i * CH
        pltpu.sync_copy(sidx_hbm.at[pl.ds(off, CH)], sidx_v)
        pltpu.sync_copy(oidx_hbm.at[pl.ds(off, CH)], oidx_v)
        s_dma = pltpu.async_copy(obj_hbm.at[sidx_v], srows_v, ssem)
        o_dma = pltpu.async_copy(obj_hbm.at[oidx_v], orows_v, osem)
        s_dma.wait()
        o_dma.wait()
        pltpu.sync_copy(srows_v, outs_hbm.at[pl.ds(off, CH)])
        pltpu.sync_copy(orows_v, outo_hbm.at[pl.ds(off, CH)])
        return carry

    lax.fori_loop(0, NCH, chunk, 0)
    off = base + NCH * CH
    pltpu.sync_copy(sidx_hbm.at[pl.ds(off, TAIL)], sidx_t)
    pltpu.sync_copy(oidx_hbm.at[pl.ds(off, TAIL)], oidx_t)
    s_dma = pltpu.async_copy(obj_hbm.at[sidx_t], srows_t, ssem)
    o_dma = pltpu.async_copy(obj_hbm.at[oidx_t], orows_t, osem)
    s_dma.wait()
    o_dma.wait()
    pltpu.sync_copy(srows_t, outs_hbm.at[pl.ds(off, TAIL)])
    pltpu.sync_copy(orows_t, outo_hbm.at[pl.ds(off, TAIL)])


@functools.partial(
    pl.kernel,
    out_type=jax.ShapeDtypeStruct((NC * OPAD, D), jnp.float32),
    mesh=_sc_mesh,
    scratch_types=[
        pltpu.VMEM_SHARED((OPAD, D), jnp.float32),
        pltpu.VMEM((CH,), jnp.int32),
        pltpu.VMEM((CH, D), jnp.float32),
        pltpu.VMEM((CH,), jnp.int32),
        pltpu.VMEM((CH, D), jnp.float32),
        pltpu.VMEM((TAIL,), jnp.int32),
        pltpu.VMEM((TAIL, D), jnp.float32),
        pltpu.VMEM((TAIL,), jnp.int32),
        pltpu.VMEM((TAIL, D), jnp.float32),
    ],
)
def _scatter_k(sidx_hbm, oidx_hbm, cs_hbm, co_hbm, zeros_hbm, outv_hbm,
               acc, sidx_v, srows_v, oidx_v, orows_v,
               sidx_t, srows_t, oidx_t, orows_t):
    c = lax.axis_index("c")
    sid = lax.axis_index("s")
    pltpu.sync_copy(zeros_hbm.at[pl.ds(0, ZR)], acc.at[pl.ds(sid * ZR, ZR)])
    plsc.subcore_barrier()

    base = (c * NS + sid) * EPW

    def chunk(i, carry):
        off = base + i * CH
        pltpu.sync_copy(sidx_hbm.at[pl.ds(off, CH)], sidx_v)
        pltpu.sync_copy(cs_hbm.at[pl.ds(off, CH)], srows_v)
        pltpu.sync_copy(srows_v, acc.at[sidx_v], add=True)
        pltpu.sync_copy(oidx_hbm.at[pl.ds(off, CH)], oidx_v)
        pltpu.sync_copy(co_hbm.at[pl.ds(off, CH)], orows_v)
        pltpu.sync_copy(orows_v, acc.at[oidx_v], add=True)
        return carry

    lax.fori_loop(0, NCH, chunk, 0)
    off = base + NCH * CH
    pltpu.sync_copy(sidx_hbm.at[pl.ds(off, TAIL)], sidx_t)
    pltpu.sync_copy(cs_hbm.at[pl.ds(off, TAIL)], srows_t)
    pltpu.sync_copy(srows_t, acc.at[sidx_t], add=True)
    pltpu.sync_copy(oidx_hbm.at[pl.ds(off, TAIL)], oidx_t)
    pltpu.sync_copy(co_hbm.at[pl.ds(off, TAIL)], orows_t)
    pltpu.sync_copy(orows_t, acc.at[oidx_t], add=True)

    plsc.subcore_barrier()
    pltpu.sync_copy(acc.at[pl.ds(sid * ZR, ZR)],
                    outv_hbm.at[pl.ds(c * OPAD + sid * ZR, ZR)])


PREP_TILE = 512
MLP_TILE = 640


def _prep_body(scal_ref, ptw_ref, ccf_ref, cnt_ref):
    i = pl.program_id(0)
    tt = scal_ref[0:1, :]
    pid = scal_ref[1:2, :]
    w = scal_ref[2:3, :]
    sif = scal_ref[3:4, :]
    oif = scal_ref[4:5, :]

    ptp = jax.nn.sigmoid(ptw_ref[...])
    sub64 = lax.broadcasted_iota(jnp.int32, (P, PREP_TILE), 0).astype(jnp.float32)
    onehot = (sub64 == pid).astype(jnp.float32)
    conf_t = jnp.dot(ptp, onehot, preferred_element_type=jnp.float32)
    conf = jnp.where(tt == 0.0, 1.0, conf_t)
    cfw = conf * w
    ccf_ref[0:1, :] = conf
    ccf_ref[1:2, :] = cfw

    lane = lax.broadcasted_iota(jnp.int32, (1, PREP_TILE), 1).astype(jnp.float32)
    batch = jnp.where(lane + i * PREP_TILE >= T, 1.0, 0.0)
    sub128 = lax.broadcasted_iota(jnp.int32, (D, PREP_TILE), 0).astype(jnp.float32)
    sub160 = lax.broadcasted_iota(jnp.int32, (2 * CROWS, PREP_TILE), 0).astype(jnp.float32)

    def hist(node_f):
        hi = jnp.floor(node_f * (1.0 / 128.0))
        lo = node_f - hi * 128.0
        hirow = hi + batch * CROWS
        a = jnp.where(sub128 == lo, cfw, 0.0)
        bm = (sub160 == hirow).astype(jnp.float32)
        return lax.dot_general(bm, a, (((1,), (1,)), ((), ())),
                               preferred_element_type=jnp.float32)

    contrib = hist(sif) + hist(oif)

    @pl.when(i == 0)
    def _():
        cnt_ref[...] = jnp.zeros_like(cnt_ref)

    cnt_ref[...] += contrib


_prep_call = pl.pallas_call(
    _prep_body,
    grid=(BT // PREP_TILE,),
    in_specs=[
        pl.BlockSpec((5, PREP_TILE), lambda i: (0, i)),
        pl.BlockSpec((1, P), lambda i: (0, 0)),
    ],
    out_specs=[
        pl.BlockSpec((2, PREP_TILE), lambda i: (0, i)),
        pl.BlockSpec((2 * CROWS, D), lambda i: (0, 0)),
    ],
    out_shape=[
        jax.ShapeDtypeStruct((2, BT), jnp.float32),
        jax.ShapeDtypeStruct((2 * CROWS, D), jnp.float32),
    ],
    compiler_params=pltpu.CompilerParams(dimension_semantics=("arbitrary",)),
)


def _mlp_body(s_ref, p_ref, o_ref, ccf_ref,
              w1s_ref, w1p_ref, w1o_ref, b1a_ref, w1b_ref, b1b_ref,
              newp_ref, cs_ref, co_ref):
    s = s_ref[...]
    pv = p_ref[...]
    o = o_ref[...]
    eye2 = (lax.broadcasted_iota(jnp.int32, (2, 2), 0)
            == lax.broadcasted_iota(jnp.int32, (2, 2), 1)).astype(jnp.float32)
    ccf_t = lax.dot_general(ccf_ref[...], eye2, (((0,), (0,)), ((), ())),
                            precision=lax.Precision.HIGHEST,
                            preferred_element_type=jnp.float32)
    conf = ccf_t[:, 0:1]
    cfw = ccf_t[:, 1:2]

    h = (jnp.dot(s, w1s_ref[...], preferred_element_type=jnp.float32)
         + jnp.dot(pv, w1p_ref[...], preferred_element_type=jnp.float32)
         + jnp.dot(o, w1o_ref[...], preferred_element_type=jnp.float32)
         + b1a_ref[...])
    h = jnp.maximum(h, 0.0)
    new_t = jnp.dot(h, w1b_ref[...], preferred_element_type=jnp.float32) + b1b_ref[...]
    new_t = jnp.maximum(new_t, 0.0)

    newp_ref[...] = new_t[:, D:2 * D] * conf
    cs_ref[...] = new_t[:, :D] * cfw
    co_ref[...] = new_t[:, 2 * D:] * cfw


def _make_mlp_call(b):
    base = b * (T // MLP_TILE)
    return pl.pallas_call(
        _mlp_body,
        grid=(T // MLP_TILE,),
        in_specs=[
            pl.BlockSpec((MLP_TILE, D), lambda i: (i, 0)),
            pl.BlockSpec((MLP_TILE, D), lambda i: (i, 0)),
            pl.BlockSpec((MLP_TILE, D), lambda i: (i, 0)),
            pl.BlockSpec((2, MLP_TILE), lambda i: (0, base + i)),
            pl.BlockSpec((D, H), lambda i: (0, 0)),
            pl.BlockSpec((D, H), lambda i: (0, 0)),
            pl.BlockSpec((D, H), lambda i: (0, 0)),
            pl.BlockSpec((1, H), lambda i: (0, 0)),
            pl.BlockSpec((H, W1B_OUT), lambda i: (0, 0)),
            pl.BlockSpec((1, W1B_OUT), lambda i: (0, 0)),
        ],
        out_specs=[
            pl.BlockSpec((MLP_TILE, D), lambda i: (i, 0)),
            pl.BlockSpec((MLP_TILE, D), lambda i: (i, 0)),
            pl.BlockSpec((MLP_TILE, D), lambda i: (i, 0)),
        ],
        out_shape=[
            jax.ShapeDtypeStruct((T, D), jnp.float32),
            jax.ShapeDtypeStruct((T, D), jnp.float32),
            jax.ShapeDtypeStruct((T, D), jnp.float32),
        ],
        compiler_params=pltpu.CompilerParams(dimension_semantics=("arbitrary",)),
    )


_mlp_calls = [_make_mlp_call(0), _make_mlp_call(1)]


OUT_TILE = 2048
OBLK = OPAD // OUT_TILE


def _out_body(pa0_ref, pa1_ref, pb0_ref, pb1_ref, cnt_ref,
              w2a_ref, b2a_ref, w2b_ref, b2b_ref, out_ref):
    i = pl.program_id(0)
    is_b0 = jnp.where(i < OBLK, 1.0, 0.0)
    pooled = ((pa0_ref[...] + pa1_ref[...]) * is_b0
              + (pb0_ref[...] + pb1_ref[...]) * (1.0 - is_b0))
    cnt = cnt_ref[...]
    denom = jnp.where(cnt > 0.0, cnt, 1.0)
    pn = pooled / denom
    h2 = jnp.maximum(
        jnp.dot(pn, w2a_ref[...], preferred_element_type=jnp.float32)
        + b2a_ref[...], 0.0)
    out_ref[...] = jnp.maximum(
        jnp.dot(h2, w2b_ref[...], preferred_element_type=jnp.float32)
        + b2b_ref[...], 0.0)


_out_call = pl.pallas_call(
    _out_body,
    grid=(B * OPAD // OUT_TILE,),
    in_specs=[
        pl.BlockSpec((OUT_TILE, D), lambda i: (i % OBLK, 0)),
        pl.BlockSpec((OUT_TILE, D), lambda i: (OBLK + i % OBLK, 0)),
        pl.BlockSpec((OUT_TILE, D), lambda i: (i % OBLK, 0)),
        pl.BlockSpec((OUT_TILE, D), lambda i: (OBLK + i % OBLK, 0)),
        pl.BlockSpec((OUT_TILE, 1), lambda i: (i, 0)),
        pl.BlockSpec((H, H), lambda i: (0, 0)),
        pl.BlockSpec((1, H), lambda i: (0, 0)),
        pl.BlockSpec((H, D), lambda i: (0, 0)),
        pl.BlockSpec((1, D), lambda i: (0, 0)),
    ],
    out_specs=pl.BlockSpec((OUT_TILE, D), lambda i: (i, 0)),
    out_shape=jax.ShapeDtypeStruct((B * OPAD, D), jnp.float32),
    compiler_params=pltpu.CompilerParams(dimension_semantics=("arbitrary",)),
)


def kernel(obj_vecs, pred_vecs, edges, pred_indicators, triplet_type,
           predicate_ids, W1a, b1a, W1b, b1b, W2a, b2a, W2b, b2b, ptw):
    s_idx = edges[:, :, 0]
    o_idx = edges[:, :, 1]
    boff = (jnp.arange(B, dtype=jnp.int32) * O)[:, None]
    sflat_g = s_idx + boff
    oflat_g = o_idx + boff
    obj_flat = obj_vecs.reshape(B * O, D)

    scal = jnp.stack([
        triplet_type.astype(jnp.float32).reshape(BT),
        predicate_ids.astype(jnp.float32).reshape(BT),
        pred_indicators.astype(jnp.float32).reshape(BT),
        s_idx.astype(jnp.float32).reshape(BT),
        o_idx.astype(jnp.float32).reshape(BT),
    ])
    ccf, cnt = _prep_call(scal, ptw.reshape(1, P))

    zeros = jnp.zeros((ZR, D), jnp.float32)
    w1s, w1p, w1o = W1a[:D], W1a[D:2 * D], W1a[2 * D:]
    b1a2 = b1a.reshape(1, H)
    b1b2 = b1b.reshape(1, W1B_OUT)

    new_ps, pvs = [], []
    for b in range(B):
        cur_s, cur_o = _gather_k(obj_flat, sflat_g[b], oflat_g[b])
        new_p_b, cs_b, co_b = _mlp_calls[b](
            cur_s, pred_vecs[b], cur_o, ccf,
            w1s, w1p, w1o, b1a2, W1b, b1b2)
        pv_b = _scatter_k(s_idx[b], o_idx[b], cs_b, co_b, zeros)
        new_ps.append(new_p_b)
        pvs.append(pv_b)

    cnt_col = cnt.reshape(B * OPAD, 1)
    new_obj = _out_call(pvs[0], pvs[0], pvs[1], pvs[1], cnt_col,
                        W2a, b2a.reshape(1, H), W2b, b2b.reshape(1, D))
    new_p = jnp.stack(new_ps)
    return new_obj.reshape(B, OPAD, D)[:, :O], new_p


# speedup vs baseline: 1.5180x; 1.0966x over previous
import functools

import jax
import jax.numpy as jnp
from jax import lax
from jax.experimental import pallas as pl
from jax.experimental.pallas import tpu as pltpu
from jax.experimental.pallas import tpu_sc as plsc

B, O, T, D, H, PO, P = 2, 10000, 160000, 128, 128, 128, 64
BT = B * T
W1B_OUT = 2 * H + PO
NC, NS = 2, 16
NW = NC * NS
EPW = T // NW
CH = 80
NCH = EPW // CH
TAIL = EPW - NCH * CH
OPAD = 10240
ZR = OPAD // NS
CROWS = OPAD // D

_sc_mesh = plsc.VectorSubcoreMesh(core_axis_name="c", subcore_axis_name="s")


@functools.partial(
    pl.kernel,
    out_type=(jax.ShapeDtypeStruct((T, D), jnp.float32),
              jax.ShapeDtypeStruct((T, D), jnp.float32)),
    mesh=_sc_mesh,
    scratch_types=[
        pltpu.VMEM((CH,), jnp.int32),
        pltpu.VMEM((CH, D), jnp.float32),
        pltpu.VMEM((CH,), jnp.int32),
        pltpu.VMEM((CH, D), jnp.float32),
        pltpu.VMEM((TAIL,), jnp.int32),
        pltpu.VMEM((TAIL, D), jnp.float32),
        pltpu.VMEM((TAIL,), jnp.int32),
        pltpu.VMEM((TAIL, D), jnp.float32),
        pltpu.SemaphoreType.DMA,
        pltpu.SemaphoreType.DMA,
    ],
)
def _gather_k(obj_hbm, sidx_hbm, oidx_hbm, outs_hbm, outo_hbm,
              sidx_v, srows_v, oidx_v, orows_v,
              sidx_t, srows_t, oidx_t, orows_t, ssem, osem):
    wid = lax.axis_index("s") * NC + lax.axis_index("c")
    base = wid * EPW

    def chunk(i, carry):
        off = base + i * CH
        pltpu.sync_copy(sidx_hbm.at[pl.ds(off, CH)], sidx_v)
        pltpu.sync_copy(oidx_hbm.at[pl.ds(off, CH)], oidx_v)
        s_dma = pltpu.async_copy(obj_hbm.at[sidx_v], srows_v, ssem)
        o_dma = pltpu.async_copy(obj_hbm.at[oidx_v], orows_v, osem)
        s_dma.wait()
        o_dma.wait()
        pltpu.sync_copy(srows_v, outs_hbm.at[pl.ds(off, CH)])
        pltpu.sync_copy(orows_v, outo_hbm.at[pl.ds(off, CH)])
        return carry

    lax.fori_loop(0, NCH, chunk, 0)
    off = base + NCH * CH
    pltpu.sync_copy(sidx_hbm.at[pl.ds(off, TAIL)], sidx_t)
    pltpu.sync_copy(oidx_hbm.at[pl.ds(off, TAIL)], oidx_t)
    s_dma = pltpu.async_copy(obj_hbm.at[sidx_t], srows_t, ssem)
    o_dma = pltpu.async_copy(obj_hbm.at[oidx_t], orows_t, osem)
    s_dma.wait()
    o_dma.wait()
    pltpu.sync_copy(srows_t, outs_hbm.at[pl.ds(off, TAIL)])
    pltpu.sync_copy(orows_t, outo_hbm.at[pl.ds(off, TAIL)])


@functools.partial(
    pl.kernel,
    out_type=jax.ShapeDtypeStruct((NC * OPAD, D), jnp.float32),
    mesh=_sc_mesh,
    scratch_types=[
        pltpu.VMEM_SHARED((OPAD, D), jnp.float32),
        pltpu.VMEM((CH,), jnp.int32),
        pltpu.VMEM((CH, D), jnp.float32),
        pltpu.VMEM((CH,), jnp.int32),
        pltpu.VMEM((CH, D), jnp.float32),
        pltpu.VMEM((TAIL,), jnp.int32),
        pltpu.VMEM((TAIL, D), jnp.float32),
        pltpu.VMEM((TAIL,), jnp.int32),
        pltpu.VMEM((TAIL, D), jnp.float32),
    ],
)
def _scatter_k(sidx_hbm, oidx_hbm, cs_hbm, co_hbm, zeros_hbm, outv_hbm,
               acc, sidx_v, srows_v, oidx_v, orows_v,
               sidx_t, srows_t, oidx_t, orows_t):
    c = lax.axis_index("c")
    sid = lax.axis_index("s")
    pltpu.sync_copy(zeros_hbm.at[pl.ds(0, ZR)], acc.at[pl.ds(sid * ZR, ZR)])
    plsc.subcore_barrier()

    base = (c * NS + sid) * EPW

    def chunk(i, carry):
        off = base + i * CH
        pltpu.sync_copy(sidx_hbm.at[pl.ds(off, CH)], sidx_v)
        pltpu.sync_copy(cs_hbm.at[pl.ds(off, CH)], srows_v)
        pltpu.sync_copy(srows_v, acc.at[sidx_v], add=True)
        pltpu.sync_copy(oidx_hbm.at[pl.ds(off, CH)], oidx_v)
        pltpu.sync_copy(co_hbm.at[pl.ds(off, CH)], orows_v)
        pltpu.sync_copy(orows_v, acc.at[oidx_v], add=True)
        return carry

    lax.fori_loop(0, NCH, chunk, 0)
    off = base + NCH * CH
    pltpu.sync_copy(sidx_hbm.at[pl.ds(off, TAIL)], sidx_t)
    pltpu.sync_copy(cs_hbm.at[pl.ds(off, TAIL)], srows_t)
    pltpu.sync_copy(srows_t, acc.at[sidx_t], add=True)
    pltpu.sync_copy(oidx_hbm.at[pl.ds(off, TAIL)], oidx_t)
    pltpu.sync_copy(co_hbm.at[pl.ds(off, TAIL)], orows_t)
    pltpu.sync_copy(orows_t, acc.at[oidx_t], add=True)

    plsc.subcore_barrier()
    pltpu.sync_copy(acc.at[pl.ds(sid * ZR, ZR)],
                    outv_hbm.at[pl.ds(c * OPAD + sid * ZR, ZR)])


PREP_TILE = 640
MLP_TILE = 640


def _prep_body(scal_ref, ptw_ref, ccf_ref, cnt_ref):
    i = pl.program_id(0)
    tt = scal_ref[0:1, :]
    pid = scal_ref[1:2, :]
    w = scal_ref[2:3, :]
    sif = scal_ref[3:4, :]
    oif = scal_ref[4:5, :]

    ptp = jax.nn.sigmoid(ptw_ref[...])
    sub64 = lax.broadcasted_iota(jnp.int32, (P, PREP_TILE), 0).astype(jnp.float32)
    onehot = (sub64 == pid).astype(jnp.float32)
    conf_t = jnp.dot(ptp, onehot, preferred_element_type=jnp.float32)
    conf = jnp.where(tt == 0.0, 1.0, conf_t)
    cfw = conf * w
    ccf_ref[0:1, :] = conf
    ccf_ref[1:2, :] = cfw

    sub128 = lax.broadcasted_iota(jnp.int32, (D, PREP_TILE), 0).astype(jnp.float32)
    sub80 = lax.broadcasted_iota(jnp.int32, (CROWS, PREP_TILE), 0).astype(jnp.float32)

    def hist(node_f):
        hi = jnp.floor(node_f * (1.0 / 128.0))
        lo = node_f - hi * 128.0
        a = jnp.where(sub128 == lo, cfw, 0.0)
        bm = (sub80 == hi).astype(jnp.float32)
        return lax.dot_general(bm, a, (((1,), (1,)), ((), ())),
                               preferred_element_type=jnp.float32)

    contrib = hist(sif) + hist(oif)

    @pl.when(i == 0)
    def _():
        cnt_ref[...] = jnp.zeros_like(cnt_ref)

    cnt_ref[...] += contrib


def _make_prep_call(b):
    base = b * (T // PREP_TILE)
    return pl.pallas_call(
        _prep_body,
        grid=(T // PREP_TILE,),
        in_specs=[
            pl.BlockSpec((5, PREP_TILE), lambda i: (0, base + i)),
            pl.BlockSpec((1, P), lambda i: (0, 0)),
        ],
        out_specs=[
            pl.BlockSpec((2, PREP_TILE), lambda i: (0, i)),
            pl.BlockSpec((CROWS, D), lambda i: (0, 0)),
        ],
        out_shape=[
            jax.ShapeDtypeStruct((2, T), jnp.float32),
            jax.ShapeDtypeStruct((CROWS, D), jnp.float32),
        ],
        compiler_params=pltpu.CompilerParams(dimension_semantics=("arbitrary",)),
    )


_prep_calls = [_make_prep_call(0), _make_prep_call(1)]


def _mlp_body(s_ref, p_ref, o_ref, ccf_ref,
              w1s_ref, w1p_ref, w1o_ref, b1a_ref, w1b_ref, b1b_ref,
              newp_ref, cs_ref, co_ref):
    s = s_ref[...]
    pv = p_ref[...]
    o = o_ref[...]
    eye2 = (lax.broadcasted_iota(jnp.int32, (2, 2), 0)
            == lax.broadcasted_iota(jnp.int32, (2, 2), 1)).astype(jnp.float32)
    ccf_t = lax.dot_general(ccf_ref[...], eye2, (((0,), (0,)), ((), ())),
                            precision=lax.Precision.HIGHEST,
                            preferred_element_type=jnp.float32)
    conf = ccf_t[:, 0:1]
    cfw = ccf_t[:, 1:2]

    h = (jnp.dot(s, w1s_ref[...], preferred_element_type=jnp.float32)
         + jnp.dot(pv, w1p_ref[...], preferred_element_type=jnp.float32)
         + jnp.dot(o, w1o_ref[...], preferred_element_type=jnp.float32)
         + b1a_ref[...])
    h = jnp.maximum(h, 0.0)
    new_t = jnp.dot(h, w1b_ref[...], preferred_element_type=jnp.float32) + b1b_ref[...]
    new_t = jnp.maximum(new_t, 0.0)

    newp_ref[...] = new_t[:, D:2 * D] * conf
    cs_ref[...] = new_t[:, :D] * cfw
    co_ref[...] = new_t[:, 2 * D:] * cfw


def _make_mlp_call(b):
    base = b * (T // MLP_TILE)
    return pl.pallas_call(
        _mlp_body,
        grid=(T // MLP_TILE,),
        in_specs=[
            pl.BlockSpec((MLP_TILE, D), lambda i: (i, 0)),
            pl.BlockSpec((MLP_TILE, D), lambda i: (base + i, 0)),
            pl.BlockSpec((MLP_TILE, D), lambda i: (i, 0)),
            pl.BlockSpec((2, MLP_TILE), lambda i: (0, i)),
            pl.BlockSpec((D, H), lambda i: (0, 0)),
            pl.BlockSpec((D, H), lambda i: (0, 0)),
            pl.BlockSpec((D, H), lambda i: (0, 0)),
            pl.BlockSpec((1, H), lambda i: (0, 0)),
            pl.BlockSpec((H, W1B_OUT), lambda i: (0, 0)),
            pl.BlockSpec((1, W1B_OUT), lambda i: (0, 0)),
        ],
        out_specs=[
            pl.BlockSpec((MLP_TILE, D), lambda i: (i, 0)),
            pl.BlockSpec((MLP_TILE, D), lambda i: (i, 0)),
            pl.BlockSpec((MLP_TILE, D), lambda i: (i, 0)),
        ],
        out_shape=[
            jax.ShapeDtypeStruct((T, D), jnp.float32),
            jax.ShapeDtypeStruct((T, D), jnp.float32),
            jax.ShapeDtypeStruct((T, D), jnp.float32),
        ],
        compiler_params=pltpu.CompilerParams(dimension_semantics=("arbitrary",)),
    )


_mlp_calls = [_make_mlp_call(0), _make_mlp_call(1)]


OUT_TILE = 2048
OBLK = OPAD // OUT_TILE


def _out_body(pa0_ref, pa1_ref, pb0_ref, pb1_ref, cnt_ref,
              w2a_ref, b2a_ref, w2b_ref, b2b_ref, out_ref):
    i = pl.program_id(0)
    is_b0 = jnp.where(i < OBLK, 1.0, 0.0)
    pooled = ((pa0_ref[...] + pa1_ref[...]) * is_b0
              + (pb0_ref[...] + pb1_ref[...]) * (1.0 - is_b0))
    cnt = cnt_ref[...]
    denom = jnp.where(cnt > 0.0, cnt, 1.0)
    pn = pooled / denom
    h2 = jnp.maximum(
        jnp.dot(pn, w2a_ref[...], preferred_element_type=jnp.float32)
        + b2a_ref[...], 0.0)
    out_ref[...] = jnp.maximum(
        jnp.dot(h2, w2b_ref[...], preferred_element_type=jnp.float32)
        + b2b_ref[...], 0.0)


_out_call = pl.pallas_call(
    _out_body,
    grid=(B * OPAD // OUT_TILE,),
    in_specs=[
        pl.BlockSpec((OUT_TILE, D), lambda i: (i % OBLK, 0)),
        pl.BlockSpec((OUT_TILE, D), lambda i: (OBLK + i % OBLK, 0)),
        pl.BlockSpec((OUT_TILE, D), lambda i: (i % OBLK, 0)),
        pl.BlockSpec((OUT_TILE, D), lambda i: (OBLK + i % OBLK, 0)),
        pl.BlockSpec((OUT_TILE, 1), lambda i: (i, 0)),
        pl.BlockSpec((H, H), lambda i: (0, 0)),
        pl.BlockSpec((1, H), lambda i: (0, 0)),
        pl.BlockSpec((H, D), lambda i: (0, 0)),
        pl.BlockSpec((1, D), lambda i: (0, 0)),
    ],
    out_specs=pl.BlockSpec((OUT_TILE, D), lambda i: (i, 0)),
    out_shape=jax.ShapeDtypeStruct((B * OPAD, D), jnp.float32),
    compiler_params=pltpu.CompilerParams(dimension_semantics=("arbitrary",)),
)


def kernel(obj_vecs, pred_vecs, edges, pred_indicators, triplet_type,
           predicate_ids, W1a, b1a, W1b, b1b, W2a, b2a, W2b, b2b, ptw):
    s_idx = edges[:, :, 0]
    o_idx = edges[:, :, 1]
    boff = (jnp.arange(B, dtype=jnp.int32) * O)[:, None]
    sflat_g = s_idx + boff
    oflat_g = o_idx + boff
    obj_flat = obj_vecs.reshape(B * O, D)

    scal = jnp.stack([
        triplet_type.astype(jnp.float32).reshape(BT),
        predicate_ids.astype(jnp.float32).reshape(BT),
        pred_indicators.astype(jnp.float32).reshape(BT),
        s_idx.astype(jnp.float32).reshape(BT),
        o_idx.astype(jnp.float32).reshape(BT),
    ])
    pred_flat = pred_vecs.reshape(BT, D)
    ptw2 = ptw.reshape(1, P)

    zeros = jnp.zeros((ZR, D), jnp.float32)
    w1s, w1p, w1o = W1a[:D], W1a[D:2 * D], W1a[2 * D:]
    b1a2 = b1a.reshape(1, H)
    b1b2 = b1b.reshape(1, W1B_OUT)

    new_ps, pvs, cnts = [], [], []
    for b in range(B):
        ccf_b, cnt_b = _prep_calls[b](scal, ptw2)
        cur_s, cur_o = _gather_k(obj_flat, sflat_g[b], oflat_g[b])
        new_p_b, cs_b, co_b = _mlp_calls[b](
            cur_s, pred_flat, cur_o, ccf_b,
            w1s, w1p, w1o, b1a2, W1b, b1b2)
        pv_b = _scatter_k(s_idx[b], o_idx[b], cs_b, co_b, zeros)
        new_ps.append(new_p_b)
        pvs.append(pv_b)
        cnts.append(cnt_b)

    cnt_col = jnp.concatenate(cnts).reshape(B * OPAD, 1)
    new_obj = _out_call(pvs[0], pvs[0], pvs[1], pvs[1], cnt_col,
                        W2a, b2a.reshape(1, H), W2b, b2b.reshape(1, D))
    new_p = jnp.stack(new_ps)
    return new_obj.reshape(B, OPAD, D)[:, :O], new_p
